# baseline (speedup 1.0000x reference)
import functools

import jax
import jax.numpy as jnp
from jax import lax
from jax.experimental import pallas as pl
from jax.experimental.pallas import tpu as pltpu
from jax.experimental.pallas import tpu_sc as plsc

ALPHA = 1.0
BETA = 10.0
NEG_RATIO = 3.0
EPS = 1e-06

N, C, H, W = 8, 3, 640, 640
HW = H * W
TOTAL = N * HW

NBINS = 1024
LANES = 16
NTILES = 32
PER_TILE = TOTAL // NTILES
TILES_PER_BATCH = NTILES // N
PART = HW // TILES_PER_BATCH
CHUNK = 12800
NCHUNK = PER_TILE // CHUNK
HISTWORDS = LANES * NBINS


_UNROLL = 8


def _sc_hist_body(pred_hbm, smap_hbm, smask_hbm, out_hbm,
                  pb0, pb1, mb0, mb1, kb0, kb1, hist, outbuf,
                  sp0, sp1, sm0, sm1, sk0, sk1):
    c = lax.axis_index("c")
    s = lax.axis_index("s")
    wid = c * 16 + s
    n = wid // TILES_PER_BATCH
    part = wid % TILES_PER_BATCH
    p_base = n * (C * HW) + part * PART
    m_base = n * HW + part * PART

    pbufs, mbufs, kbufs = (pb0, pb1), (mb0, mb1), (kb0, kb1)
    sems = ((sp0, sm0, sk0), (sp1, sm1, sk1))

    def start(ci):
        b = ci % 2
        return (
            pltpu.async_copy(pred_hbm.at[pl.ds(p_base + ci * CHUNK, CHUNK)],
                             pbufs[b], sems[b][0]),
            pltpu.async_copy(smap_hbm.at[pl.ds(m_base + ci * CHUNK, CHUNK)],
                             mbufs[b], sems[b][1]),
            pltpu.async_copy(smask_hbm.at[pl.ds(m_base + ci * CHUNK, CHUNK)],
                             kbufs[b], sems[b][2]),
        )

    pending = start(0)

    def zero_body(i, carry):
        o = i * (16 * _UNROLL)
        for j in range(_UNROLL):
            hist[pl.ds(o + j * 16, 16)] = jnp.zeros((16,), jnp.float32)
        return carry
    lax.fori_loop(0, HISTWORDS // (16 * _UNROLL), zero_body, 0)

    lane_off = lax.iota(jnp.int32, 16) * NBINS

    for ci in range(NCHUNK):
        nxt = start(ci + 1) if ci + 1 < NCHUNK else None
        for h in pending:
            h.wait()
        pbuf, mbuf, kbuf = pbufs[ci % 2], mbufs[ci % 2], kbufs[ci % 2]

        def vec_body(i, inner, pbuf=pbuf, mbuf=mbuf, kbuf=kbuf):
            o0 = i * (16 * _UNROLL)
            for j in range(_UNROLL):
                o = o0 + j * 16
                p = pbuf[pl.ds(o, 16)]
                sm = mbuf[pl.ds(o, 16)]
                sk = kbuf[pl.ds(o, 16)]
                neg = sk * (1.0 - sm)
                b = (p * float(NBINS)).astype(jnp.int32)
                plsc.addupdate_scatter(hist, [lane_off + b], neg)
            return inner
        lax.fori_loop(0, CHUNK // (16 * _UNROLL), vec_body, 0)
        pending = nxt

    def red_body(g, carry):
        acc = jnp.zeros((16,), jnp.float32)
        for l in range(LANES):
            acc = acc + hist[pl.ds(l * NBINS + g * 16, 16)]
        outbuf[pl.ds(g * 16, 16)] = acc
        return carry
    lax.fori_loop(0, NBINS // 16, red_body, 0)

    pltpu.sync_copy(outbuf, out_hbm.at[wid])


@functools.lru_cache(maxsize=1)
def _sc_hist():
    return functools.partial(
        pl.kernel,
        out_type=jax.ShapeDtypeStruct((NTILES, NBINS), jnp.float32),
        scratch_types=(
            [pltpu.VMEM((CHUNK,), jnp.float32) for _ in range(6)]
            + [pltpu.VMEM((HISTWORDS,), jnp.float32),
               pltpu.VMEM((NBINS,), jnp.float32)]
            + [pltpu.SemaphoreType.DMA for _ in range(6)]
        ),
        mesh=plsc.VectorSubcoreMesh(core_axis_name="c", subcore_axis_name="s"),
        compiler_params=pltpu.CompilerParams(
            needs_layout_passes=False, use_tc_tiling_on_sc=False),
    )(_sc_hist_body)


_BS = 400


def _reduce_body(pred_ref, smap_ref, smask_ref, tmap_ref, tmask_ref, acc_ref):
    i = pl.program_id(0)
    p0 = pred_ref[:, 0, :, :]
    p1 = pred_ref[:, 1, :, :]
    p2 = pred_ref[:, 2, :, :]
    smap = smap_ref[...]
    smask = smask_ref[...]
    tmap = tmap_ref[...]
    tmask = tmask_ref[...]

    pos = jnp.floor(smap * smask)
    neg = jnp.floor((1.0 - smap) * smask)
    log_p = jnp.maximum(jnp.log(p0), -100.0)
    log_1p = jnp.maximum(jnp.log(1.0 - p0), -100.0)
    loss = -(smap * log_p + (1.0 - smap) * log_1p)

    sums = (
        jnp.sum(pos),
        jnp.sum(neg),
        jnp.sum(loss * pos),
        jnp.sum(loss * neg),
        jnp.sum(jnp.abs(p1 - tmap) * tmask),
        jnp.sum(tmask),
        jnp.sum(p2 * smap * smask),
        jnp.sum(p2 * smask),
        jnp.sum(smap * smask),
    )
    lane = lax.broadcasted_iota(jnp.int32, (1, 128), 1)
    packed = jnp.zeros((1, 128), jnp.float32)
    for j, v in enumerate(sums):
        packed = packed + jnp.where(lane == j, v, 0.0)

    @pl.when(i == 0)
    def _():
        acc_ref[...] = packed

    @pl.when(i > 0)
    def _():
        acc_ref[...] = acc_ref[...] + packed


def _finalize_body(acc_ref, hist_ref, out_ref):
    pos_cnt = acc_ref[0, 0]
    neg_tot = acc_ref[0, 1]
    pos_loss = acc_ref[0, 2]
    neg_loss = acc_ref[0, 3]
    l1_num = acc_ref[0, 4]
    tmask_sum = acc_ref[0, 5]
    inter = acc_ref[0, 6]
    u1 = acc_ref[0, 7]
    u2 = acc_ref[0, 8]

    counts = jnp.sum(hist_ref[...], axis=0, keepdims=True)
    ge = (lax.broadcasted_iota(jnp.int32, (NBINS, NBINS), 0) >=
          lax.broadcasted_iota(jnp.int32, (NBINS, NBINS), 1)).astype(jnp.float32)
    T = jax.lax.dot_general(counts, ge, (((1,), (0,)), ((), ())),
                            preferred_element_type=jnp.float32)
    above = T - counts
    k = jnp.minimum(neg_tot, pos_cnt * NEG_RATIO)
    included = jnp.clip(k - above, 0.0, counts)
    excluded = counts - included
    centers = (lax.broadcasted_iota(jnp.int32, (1, NBINS), 1).astype(jnp.float32)
               + 0.5) / NBINS
    mid_loss = -jnp.log(1.0 - centers)
    excl_est = jnp.sum(excluded * mid_loss)
    top_sum = jnp.where(k > 0.5,
                        jnp.clip(neg_loss - excl_est, 0.0, neg_loss),
                        0.0)

    loss_shrink = (pos_loss + top_sum) / (pos_cnt + k + EPS)
    loss_thresh = l1_num / (tmask_sum + EPS)
    loss_dice = 1.0 - 2.0 * inter / (u1 + u2 + EPS)
    loss_all = ALPHA * loss_shrink + BETA * loss_thresh + loss_dice

    lane = lax.broadcasted_iota(jnp.int32, (1, 128), 1)
    out = jnp.where(lane == 0, loss_all, 0.0)
    out = out + jnp.where(lane == 1, loss_shrink, 0.0)
    out = out + jnp.where(lane == 2, loss_thresh, 0.0)
    out = out + jnp.where(lane == 3, loss_dice, 0.0)
    out_ref[...] = out


def kernel(pred, shrink_map, shrink_mask, threshold_map, threshold_mask):
    pred4 = pred.reshape(N, C, HW // 128, 128)
    smap3 = shrink_map.reshape(N, HW // 128, 128)
    smask3 = shrink_mask.reshape(N, HW // 128, 128)
    tmap3 = threshold_map.reshape(N, HW // 128, 128)
    tmask3 = threshold_mask.reshape(N, HW // 128, 128)

    rows = HW // 128
    grid = rows // _BS

    acc = pl.pallas_call(
        _reduce_body,
        grid=(grid,),
        in_specs=[
            pl.BlockSpec((N, C, _BS, 128), lambda i: (0, 0, i, 0)),
            pl.BlockSpec((N, _BS, 128), lambda i: (0, i, 0)),
            pl.BlockSpec((N, _BS, 128), lambda i: (0, i, 0)),
            pl.BlockSpec((N, _BS, 128), lambda i: (0, i, 0)),
            pl.BlockSpec((N, _BS, 128), lambda i: (0, i, 0)),
        ],
        out_specs=pl.BlockSpec((1, 128), lambda i: (0, 0)),
        out_shape=jax.ShapeDtypeStruct((1, 128), jnp.float32),
    )(pred4, smap3, smask3, tmap3, tmask3)

    hist = _sc_hist()(pred.reshape(-1), shrink_map.reshape(-1),
                      shrink_mask.reshape(-1))

    out = pl.pallas_call(
        _finalize_body,
        out_shape=jax.ShapeDtypeStruct((1, 128), jnp.float32),
    )(acc, hist)

    return (out[0, 0], out[0, 1], out[0, 2], out[0, 3])


# speedup vs baseline: 1.3103x; 1.3103x over previous
import functools

import jax
import jax.numpy as jnp
from jax import lax
from jax.experimental import pallas as pl
from jax.experimental.pallas import tpu as pltpu
from jax.experimental.pallas import tpu_sc as plsc

ALPHA = 1.0
BETA = 10.0
NEG_RATIO = 3.0
EPS = 1e-06

N, C, H, W = 8, 3, 640, 640
HW = H * W

NBINS = 1024
LANES = 16
NTILES = 32
TILES_PER_BATCH = NTILES // N
ROWS_PER_TILE = H // TILES_PER_BATCH
CHUNK_ROWS = 20
NCHUNK = ROWS_PER_TILE // CHUNK_ROWS
VECS_PER_ROW = W // LANES
HISTWORDS = LANES * NBINS


def _sc_hist_body(pred_hbm, smap_hbm, smask_hbm, out_hbm,
                  pb0, pb1, mb0, mb1, kb0, kb1, hist, outbuf,
                  sp0, sp1, sm0, sm1, sk0, sk1):
    c = lax.axis_index("c")
    s = lax.axis_index("s")
    wid = c * 16 + s
    n = wid // TILES_PER_BATCH
    part = wid % TILES_PER_BATCH
    row0 = part * ROWS_PER_TILE

    pbufs, mbufs, kbufs = (pb0, pb1), (mb0, mb1), (kb0, kb1)
    sems = ((sp0, sm0, sk0), (sp1, sm1, sk1))

    def start(ci):
        b = ci % 2
        r = row0 + ci * CHUNK_ROWS
        return (
            pltpu.async_copy(pred_hbm.at[n, 0, pl.ds(r, CHUNK_ROWS), :],
                             pbufs[b], sems[b][0]),
            pltpu.async_copy(smap_hbm.at[n, pl.ds(r, CHUNK_ROWS), :],
                             mbufs[b], sems[b][1]),
            pltpu.async_copy(smask_hbm.at[n, pl.ds(r, CHUNK_ROWS), :],
                             kbufs[b], sems[b][2]),
        )

    pending = start(0)

    def zero_body(i, carry):
        o = i * 128
        for j in range(8):
            hist[pl.ds(o + j * 16, 16)] = jnp.zeros((16,), jnp.float32)
        return carry
    lax.fori_loop(0, HISTWORDS // 128, zero_body, 0)

    lane_off = lax.iota(jnp.int32, 16) * NBINS

    for ci in range(NCHUNK):
        nxt = start(ci + 1) if ci + 1 < NCHUNK else None
        for h in pending:
            h.wait()
        pbuf, mbuf, kbuf = pbufs[ci % 2], mbufs[ci % 2], kbufs[ci % 2]

        def row_body(r, inner, pbuf=pbuf, mbuf=mbuf, kbuf=kbuf):
            for j in range(VECS_PER_ROW):
                o = j * 16
                p = pbuf[r, pl.ds(o, 16)]
                sm = mbuf[r, pl.ds(o, 16)]
                sk = kbuf[r, pl.ds(o, 16)]
                neg = sk * (1.0 - sm)
                b = (p * float(NBINS)).astype(jnp.int32)
                plsc.addupdate_scatter(hist, [lane_off + b], neg)
            return inner
        lax.fori_loop(0, CHUNK_ROWS, row_body, 0)
        pending = nxt

    def red_body(g, carry):
        acc = jnp.zeros((16,), jnp.float32)
        for l in range(LANES):
            acc = acc + hist[pl.ds(l * NBINS + g * 16, 16)]
        outbuf[pl.ds(g * 16, 16)] = acc
        return carry
    lax.fori_loop(0, NBINS // 16, red_body, 0)

    pltpu.sync_copy(outbuf, out_hbm.at[wid])


@functools.lru_cache(maxsize=1)
def _sc_hist():
    return functools.partial(
        pl.kernel,
        out_type=jax.ShapeDtypeStruct((NTILES, NBINS), jnp.float32),
        scratch_types=(
            [pltpu.VMEM((CHUNK_ROWS, W), jnp.float32) for _ in range(6)]
            + [pltpu.VMEM((HISTWORDS,), jnp.float32),
               pltpu.VMEM((NBINS,), jnp.float32)]
            + [pltpu.SemaphoreType.DMA for _ in range(6)]
        ),
        mesh=plsc.VectorSubcoreMesh(core_axis_name="c", subcore_axis_name="s"),
        compiler_params=pltpu.CompilerParams(
            needs_layout_passes=False, use_tc_tiling_on_sc=False),
    )(_sc_hist_body)


_BH = 80


def _reduce_body(pred_ref, smap_ref, smask_ref, tmap_ref, tmask_ref, acc_ref):
    i = pl.program_id(0)
    p0 = pred_ref[:, 0, :, :]
    p1 = pred_ref[:, 1, :, :]
    p2 = pred_ref[:, 2, :, :]
    smap = smap_ref[...]
    smask = smask_ref[...]
    tmap = tmap_ref[...]
    tmask = tmask_ref[...]

    pos = jnp.floor(smap * smask)
    neg = jnp.floor((1.0 - smap) * smask)
    log_p = jnp.maximum(jnp.log(p0), -100.0)
    log_1p = jnp.maximum(jnp.log(1.0 - p0), -100.0)
    loss = -(smap * log_p + (1.0 - smap) * log_1p)

    sums = (
        jnp.sum(pos),
        jnp.sum(neg),
        jnp.sum(loss * pos),
        jnp.sum(loss * neg),
        jnp.sum(jnp.abs(p1 - tmap) * tmask),
        jnp.sum(tmask),
        jnp.sum(p2 * smap * smask),
        jnp.sum(p2 * smask),
        jnp.sum(smap * smask),
    )
    lane = lax.broadcasted_iota(jnp.int32, (1, 128), 1)
    packed = jnp.zeros((1, 128), jnp.float32)
    for j, v in enumerate(sums):
        packed = packed + jnp.where(lane == j, v, 0.0)

    @pl.when(i == 0)
    def _():
        acc_ref[...] = packed

    @pl.when(i > 0)
    def _():
        acc_ref[...] = acc_ref[...] + packed


def _finalize_body(acc_ref, hist_ref, out_ref):
    pos_cnt = acc_ref[0, 0]
    neg_tot = acc_ref[0, 1]
    pos_loss = acc_ref[0, 2]
    neg_loss = acc_ref[0, 3]
    l1_num = acc_ref[0, 4]
    tmask_sum = acc_ref[0, 5]
    inter = acc_ref[0, 6]
    u1 = acc_ref[0, 7]
    u2 = acc_ref[0, 8]

    counts = jnp.sum(hist_ref[...], axis=0, keepdims=True)
    ge = (lax.broadcasted_iota(jnp.int32, (NBINS, NBINS), 0) >=
          lax.broadcasted_iota(jnp.int32, (NBINS, NBINS), 1)).astype(jnp.float32)
    T = jax.lax.dot_general(counts, ge, (((1,), (0,)), ((), ())),
                            preferred_element_type=jnp.float32)
    above = T - counts
    k = jnp.minimum(neg_tot, pos_cnt * NEG_RATIO)
    included = jnp.clip(k - above, 0.0, counts)
    excluded = counts - included
    centers = (lax.broadcasted_iota(jnp.int32, (1, NBINS), 1).astype(jnp.float32)
               + 0.5) / NBINS
    mid_loss = -jnp.log(1.0 - centers)
    excl_est = jnp.sum(excluded * mid_loss)
    top_sum = jnp.where(k > 0.5,
                        jnp.clip(neg_loss - excl_est, 0.0, neg_loss),
                        0.0)

    loss_shrink = (pos_loss + top_sum) / (pos_cnt + k + EPS)
    loss_thresh = l1_num / (tmask_sum + EPS)
    loss_dice = 1.0 - 2.0 * inter / (u1 + u2 + EPS)
    loss_all = ALPHA * loss_shrink + BETA * loss_thresh + loss_dice

    lane = lax.broadcasted_iota(jnp.int32, (1, 128), 1)
    out = jnp.where(lane == 0, loss_all, 0.0)
    out = out + jnp.where(lane == 1, loss_shrink, 0.0)
    out = out + jnp.where(lane == 2, loss_thresh, 0.0)
    out = out + jnp.where(lane == 3, loss_dice, 0.0)
    out_ref[...] = out


def kernel(pred, shrink_map, shrink_mask, threshold_map, threshold_mask):
    grid = H // _BH

    acc = pl.pallas_call(
        _reduce_body,
        grid=(grid,),
        in_specs=[
            pl.BlockSpec((N, C, _BH, W), lambda i: (0, 0, i, 0)),
            pl.BlockSpec((N, _BH, W), lambda i: (0, i, 0)),
            pl.BlockSpec((N, _BH, W), lambda i: (0, i, 0)),
            pl.BlockSpec((N, _BH, W), lambda i: (0, i, 0)),
            pl.BlockSpec((N, _BH, W), lambda i: (0, i, 0)),
        ],
        out_specs=pl.BlockSpec((1, 128), lambda i: (0, 0)),
        out_shape=jax.ShapeDtypeStruct((1, 128), jnp.float32),
    )(pred, shrink_map, shrink_mask, threshold_map, threshold_mask)

    hist = _sc_hist()(pred, shrink_map, shrink_mask)

    out = pl.pallas_call(
        _finalize_body,
        out_shape=jax.ShapeDtypeStruct((1, 128), jnp.float32),
    )(acc, hist)

    return (out[0, 0], out[0, 1], out[0, 2], out[0, 3])


# speedup vs baseline: 2.0291x; 1.5486x over previous
import functools

import jax
import jax.numpy as jnp
from jax import lax
from jax.experimental import pallas as pl
from jax.experimental.pallas import tpu as pltpu
from jax.experimental.pallas import tpu_sc as plsc

ALPHA = 1.0
BETA = 10.0
NEG_RATIO = 3.0
EPS = 1e-06

N, C, H, W = 8, 3, 640, 640
HW = H * W

NBINS = 1024
LANES = 16
NTILES = 32
TILES_PER_BATCH = NTILES // N
ROWS_PER_TILE = H // TILES_PER_BATCH
CHUNK_ROWS = 16
NCHUNK = ROWS_PER_TILE // CHUNK_ROWS
VECS_PER_ROW = W // LANES
HISTWORDS = LANES * NBINS


def _sc_hist_body(pred_hbm, smap_hbm, smask_hbm, out_hbm,
                  pb0, pb1, mb0, mb1, kb0, kb1, hist, outbuf,
                  sp0, sp1, sm0, sm1, sk0, sk1):
    c = lax.axis_index("c")
    s = lax.axis_index("s")
    wid = c * 16 + s
    n = wid // TILES_PER_BATCH
    part = wid % TILES_PER_BATCH
    row0 = part * ROWS_PER_TILE

    pbufs, mbufs, kbufs = (pb0, pb1), (mb0, mb1), (kb0, kb1)
    sems = ((sp0, sm0, sk0), (sp1, sm1, sk1))

    def start(ci):
        b = ci % 2
        r = row0 + ci * CHUNK_ROWS
        return (
            pltpu.async_copy(pred_hbm.at[n, 0, pl.ds(r, CHUNK_ROWS), :],
                             pbufs[b], sems[b][0]),
            pltpu.async_copy(smap_hbm.at[n, pl.ds(r, CHUNK_ROWS), :],
                             mbufs[b], sems[b][1]),
            pltpu.async_copy(smask_hbm.at[n, pl.ds(r, CHUNK_ROWS), :],
                             kbufs[b], sems[b][2]),
        )

    pending = start(0)

    def zero_body(i, carry):
        o = i * 128
        for j in range(8):
            hist[pl.ds(o + j * 16, 16)] = jnp.zeros((16,), jnp.float32)
        return carry
    lax.fori_loop(0, HISTWORDS // 128, zero_body, 0)

    lane_off = lax.iota(jnp.int32, 16) * NBINS

    for ci in range(NCHUNK):
        nxt = start(ci + 1) if ci + 1 < NCHUNK else None
        for h in pending:
            h.wait()
        pbuf, mbuf, kbuf = pbufs[ci % 2], mbufs[ci % 2], kbufs[ci % 2]

        def row_body(r, inner, pbuf=pbuf, mbuf=mbuf, kbuf=kbuf):
            for j in range(VECS_PER_ROW):
                o = j * 16
                p = pbuf[r, pl.ds(o, 16)]
                sm = mbuf[r, pl.ds(o, 16)]
                sk = kbuf[r, pl.ds(o, 16)]
                neg = sk * (1.0 - sm)
                b = (p * float(NBINS)).astype(jnp.int32)
                plsc.addupdate_scatter(hist, [lane_off + b], neg)
            return inner
        lax.fori_loop(0, CHUNK_ROWS, row_body, 0)
        pending = nxt

    def red_body(g, carry):
        acc = jnp.zeros((16,), jnp.float32)
        for l in range(LANES):
            acc = acc + hist[pl.ds(l * NBINS + g * 16, 16)]
        outbuf[pl.ds(g * 16, 16)] = acc
        return carry
    lax.fori_loop(0, NBINS // 16, red_body, 0)

    pltpu.sync_copy(outbuf, out_hbm.at[wid])


@functools.lru_cache(maxsize=1)
def _sc_hist():
    return functools.partial(
        pl.kernel,
        out_type=jax.ShapeDtypeStruct((NTILES, NBINS), jnp.float32),
        scratch_types=(
            [pltpu.VMEM((CHUNK_ROWS, W), jnp.float32) for _ in range(6)]
            + [pltpu.VMEM((HISTWORDS,), jnp.float32),
               pltpu.VMEM((NBINS,), jnp.float32)]
            + [pltpu.SemaphoreType.DMA for _ in range(6)]
        ),
        mesh=plsc.VectorSubcoreMesh(core_axis_name="c", subcore_axis_name="s"),
        compiler_params=pltpu.CompilerParams(
            needs_layout_passes=False),
    )(_sc_hist_body)


_BH = 80


def _reduce_body(pred_ref, smap_ref, smask_ref, tmap_ref, tmask_ref, acc_ref):
    i = pl.program_id(0)
    p0 = pred_ref[:, 0, :, :]
    p1 = pred_ref[:, 1, :, :]
    p2 = pred_ref[:, 2, :, :]
    smap = smap_ref[...]
    smask = smask_ref[...]
    tmap = tmap_ref[...]
    tmask = tmask_ref[...]

    pos = jnp.floor(smap * smask)
    neg = jnp.floor((1.0 - smap) * smask)
    log_p = jnp.maximum(jnp.log(p0), -100.0)
    log_1p = jnp.maximum(jnp.log(1.0 - p0), -100.0)
    loss = -(smap * log_p + (1.0 - smap) * log_1p)

    sums = (
        jnp.sum(pos),
        jnp.sum(neg),
        jnp.sum(loss * pos),
        jnp.sum(loss * neg),
        jnp.sum(jnp.abs(p1 - tmap) * tmask),
        jnp.sum(tmask),
        jnp.sum(p2 * smap * smask),
        jnp.sum(p2 * smask),
        jnp.sum(smap * smask),
    )
    lane = lax.broadcasted_iota(jnp.int32, (1, 128), 1)
    packed = jnp.zeros((1, 128), jnp.float32)
    for j, v in enumerate(sums):
        packed = packed + jnp.where(lane == j, v, 0.0)

    @pl.when(i == 0)
    def _():
        acc_ref[...] = packed

    @pl.when(i > 0)
    def _():
        acc_ref[...] = acc_ref[...] + packed


def _finalize_body(acc_ref, hist_ref, out_ref):
    pos_cnt = acc_ref[0, 0]
    neg_tot = acc_ref[0, 1]
    pos_loss = acc_ref[0, 2]
    neg_loss = acc_ref[0, 3]
    l1_num = acc_ref[0, 4]
    tmask_sum = acc_ref[0, 5]
    inter = acc_ref[0, 6]
    u1 = acc_ref[0, 7]
    u2 = acc_ref[0, 8]

    counts = jnp.sum(hist_ref[...], axis=0, keepdims=True)
    ge = (lax.broadcasted_iota(jnp.int32, (NBINS, NBINS), 0) >=
          lax.broadcasted_iota(jnp.int32, (NBINS, NBINS), 1)).astype(jnp.float32)
    T = jax.lax.dot_general(counts, ge, (((1,), (0,)), ((), ())),
                            preferred_element_type=jnp.float32)
    above = T - counts
    k = jnp.minimum(neg_tot, pos_cnt * NEG_RATIO)
    included = jnp.clip(k - above, 0.0, counts)
    excluded = counts - included
    centers = (lax.broadcasted_iota(jnp.int32, (1, NBINS), 1).astype(jnp.float32)
               + 0.5) / NBINS
    mid_loss = -jnp.log(1.0 - centers)
    excl_est = jnp.sum(excluded * mid_loss)
    top_sum = jnp.where(k > 0.5,
                        jnp.clip(neg_loss - excl_est, 0.0, neg_loss),
                        0.0)

    loss_shrink = (pos_loss + top_sum) / (pos_cnt + k + EPS)
    loss_thresh = l1_num / (tmask_sum + EPS)
    loss_dice = 1.0 - 2.0 * inter / (u1 + u2 + EPS)
    loss_all = ALPHA * loss_shrink + BETA * loss_thresh + loss_dice

    lane = lax.broadcasted_iota(jnp.int32, (1, 128), 1)
    out = jnp.where(lane == 0, loss_all, 0.0)
    out = out + jnp.where(lane == 1, loss_shrink, 0.0)
    out = out + jnp.where(lane == 2, loss_thresh, 0.0)
    out = out + jnp.where(lane == 3, loss_dice, 0.0)
    out_ref[...] = out


def kernel(pred, shrink_map, shrink_mask, threshold_map, threshold_mask):
    grid = H // _BH

    acc = pl.pallas_call(
        _reduce_body,
        grid=(grid,),
        in_specs=[
            pl.BlockSpec((N, C, _BH, W), lambda i: (0, 0, i, 0)),
            pl.BlockSpec((N, _BH, W), lambda i: (0, i, 0)),
            pl.BlockSpec((N, _BH, W), lambda i: (0, i, 0)),
            pl.BlockSpec((N, _BH, W), lambda i: (0, i, 0)),
            pl.BlockSpec((N, _BH, W), lambda i: (0, i, 0)),
        ],
        out_specs=pl.BlockSpec((1, 128), lambda i: (0, 0)),
        out_shape=jax.ShapeDtypeStruct((1, 128), jnp.float32),
    )(pred, shrink_map, shrink_mask, threshold_map, threshold_mask)

    hist = _sc_hist()(pred, shrink_map, shrink_mask)

    out = pl.pallas_call(
        _finalize_body,
        out_shape=jax.ShapeDtypeStruct((1, 128), jnp.float32),
    )(acc, hist)

    return (out[0, 0], out[0, 1], out[0, 2], out[0, 3])


# speedup vs baseline: 2.0433x; 1.0070x over previous
import functools

import jax
import jax.numpy as jnp
from jax import lax
from jax.experimental import pallas as pl
from jax.experimental.pallas import tpu as pltpu
from jax.experimental.pallas import tpu_sc as plsc

ALPHA = 1.0
BETA = 10.0
NEG_RATIO = 3.0
EPS = 1e-06

N, C, H, W = 8, 3, 640, 640
HW = H * W

NBINS = 1024
LANES = 16
NTILES = 32
TILES_PER_BATCH = NTILES // N
ROWS_PER_TILE = H // TILES_PER_BATCH
CHUNK_ROWS = 16
NCHUNK = ROWS_PER_TILE // CHUNK_ROWS
VECS_PER_ROW = W // LANES
LANE_STRIDE = NBINS + 1
HISTWORDS = LANES * LANE_STRIDE


def _sc_hist_body(pred_hbm, smap_hbm, smask_hbm, out_hbm,
                  pb0, pb1, mb0, mb1, kb0, kb1, hist, outbuf,
                  sp0, sp1, sm0, sm1, sk0, sk1):
    c = lax.axis_index("c")
    s = lax.axis_index("s")
    wid = c * 16 + s
    n = wid // TILES_PER_BATCH
    part = wid % TILES_PER_BATCH
    row0 = part * ROWS_PER_TILE

    pbufs, mbufs, kbufs = (pb0, pb1), (mb0, mb1), (kb0, kb1)
    sems = ((sp0, sm0, sk0), (sp1, sm1, sk1))

    def start(ci):
        b = ci % 2
        r = row0 + ci * CHUNK_ROWS
        return (
            pltpu.async_copy(pred_hbm.at[n, 0, pl.ds(r, CHUNK_ROWS), :],
                             pbufs[b], sems[b][0]),
            pltpu.async_copy(smap_hbm.at[n, pl.ds(r, CHUNK_ROWS), :],
                             mbufs[b], sems[b][1]),
            pltpu.async_copy(smask_hbm.at[n, pl.ds(r, CHUNK_ROWS), :],
                             kbufs[b], sems[b][2]),
        )

    pending = start(0)

    def zero_body(i, carry):
        o = i * 128
        for j in range(8):
            hist[pl.ds(o + j * 16, 16)] = jnp.zeros((16,), jnp.float32)
        return carry
    lax.fori_loop(0, (HISTWORDS + 127) // 128, zero_body, 0)

    lane_off = lax.iota(jnp.int32, 16) * LANE_STRIDE

    for ci in range(NCHUNK):
        nxt = start(ci + 1) if ci + 1 < NCHUNK else None
        for h in pending:
            h.wait()
        pbuf, mbuf, kbuf = pbufs[ci % 2], mbufs[ci % 2], kbufs[ci % 2]

        def row_body(r, inner, pbuf=pbuf, mbuf=mbuf, kbuf=kbuf):
            for j in range(VECS_PER_ROW):
                o = j * 16
                p = pbuf[r, pl.ds(o, 16)]
                sm = mbuf[r, pl.ds(o, 16)]
                sk = kbuf[r, pl.ds(o, 16)]
                neg = sk * (1.0 - sm)
                b = (p * float(NBINS)).astype(jnp.int32)
                plsc.addupdate_scatter(hist, [lane_off + b], neg)
            return inner
        lax.fori_loop(0, CHUNK_ROWS, row_body, 0)
        pending = nxt

    def red_body(g, carry):
        acc = jnp.zeros((16,), jnp.float32)
        for l in range(LANES):
            acc = acc + hist[pl.ds(l * LANE_STRIDE + g * 16, 16)]
        outbuf[pl.ds(g * 16, 16)] = acc
        return carry
    lax.fori_loop(0, NBINS // 16, red_body, 0)

    pltpu.sync_copy(outbuf, out_hbm.at[wid])


@functools.lru_cache(maxsize=1)
def _sc_hist():
    return functools.partial(
        pl.kernel,
        out_type=jax.ShapeDtypeStruct((NTILES, NBINS), jnp.float32),
        scratch_types=(
            [pltpu.VMEM((CHUNK_ROWS, W), jnp.float32) for _ in range(6)]
            + [pltpu.VMEM((HISTWORDS + 128,), jnp.float32),
               pltpu.VMEM((NBINS,), jnp.float32)]
            + [pltpu.SemaphoreType.DMA for _ in range(6)]
        ),
        mesh=plsc.VectorSubcoreMesh(core_axis_name="c", subcore_axis_name="s"),
        compiler_params=pltpu.CompilerParams(
            needs_layout_passes=False),
    )(_sc_hist_body)


_BH = 80


def _reduce_body(pred_ref, smap_ref, smask_ref, tmap_ref, tmask_ref, acc_ref):
    i = pl.program_id(0)
    p0 = pred_ref[:, 0, :, :]
    p1 = pred_ref[:, 1, :, :]
    p2 = pred_ref[:, 2, :, :]
    smap = smap_ref[...]
    smask = smask_ref[...]
    tmap = tmap_ref[...]
    tmask = tmask_ref[...]

    pos = jnp.floor(smap * smask)
    neg = jnp.floor((1.0 - smap) * smask)
    log_p = jnp.maximum(jnp.log(p0), -100.0)
    log_1p = jnp.maximum(jnp.log(1.0 - p0), -100.0)
    loss = -(smap * log_p + (1.0 - smap) * log_1p)

    sums = (
        jnp.sum(pos),
        jnp.sum(neg),
        jnp.sum(loss * pos),
        jnp.sum(loss * neg),
        jnp.sum(jnp.abs(p1 - tmap) * tmask),
        jnp.sum(tmask),
        jnp.sum(p2 * smap * smask),
        jnp.sum(p2 * smask),
        jnp.sum(smap * smask),
    )
    lane = lax.broadcasted_iota(jnp.int32, (1, 128), 1)
    packed = jnp.zeros((1, 128), jnp.float32)
    for j, v in enumerate(sums):
        packed = packed + jnp.where(lane == j, v, 0.0)

    @pl.when(i == 0)
    def _():
        acc_ref[...] = packed

    @pl.when(i > 0)
    def _():
        acc_ref[...] = acc_ref[...] + packed


def _finalize_body(acc_ref, hist_ref, out_ref):
    pos_cnt = acc_ref[0, 0]
    neg_tot = acc_ref[0, 1]
    pos_loss = acc_ref[0, 2]
    neg_loss = acc_ref[0, 3]
    l1_num = acc_ref[0, 4]
    tmask_sum = acc_ref[0, 5]
    inter = acc_ref[0, 6]
    u1 = acc_ref[0, 7]
    u2 = acc_ref[0, 8]

    counts = jnp.sum(hist_ref[...], axis=0, keepdims=True)
    ge = (lax.broadcasted_iota(jnp.int32, (NBINS, NBINS), 0) >=
          lax.broadcasted_iota(jnp.int32, (NBINS, NBINS), 1)).astype(jnp.float32)
    T = jax.lax.dot_general(counts, ge, (((1,), (0,)), ((), ())),
                            preferred_element_type=jnp.float32)
    above = T - counts
    k = jnp.minimum(neg_tot, pos_cnt * NEG_RATIO)
    included = jnp.clip(k - above, 0.0, counts)
    excluded = counts - included
    centers = (lax.broadcasted_iota(jnp.int32, (1, NBINS), 1).astype(jnp.float32)
               + 0.5) / NBINS
    mid_loss = -jnp.log(1.0 - centers)
    excl_est = jnp.sum(excluded * mid_loss)
    top_sum = jnp.where(k > 0.5,
                        jnp.clip(neg_loss - excl_est, 0.0, neg_loss),
                        0.0)

    loss_shrink = (pos_loss + top_sum) / (pos_cnt + k + EPS)
    loss_thresh = l1_num / (tmask_sum + EPS)
    loss_dice = 1.0 - 2.0 * inter / (u1 + u2 + EPS)
    loss_all = ALPHA * loss_shrink + BETA * loss_thresh + loss_dice

    lane = lax.broadcasted_iota(jnp.int32, (1, 128), 1)
    out = jnp.where(lane == 0, loss_all, 0.0)
    out = out + jnp.where(lane == 1, loss_shrink, 0.0)
    out = out + jnp.where(lane == 2, loss_thresh, 0.0)
    out = out + jnp.where(lane == 3, loss_dice, 0.0)
    out_ref[...] = out


def kernel(pred, shrink_map, shrink_mask, threshold_map, threshold_mask):
    grid = H // _BH

    acc = pl.pallas_call(
        _reduce_body,
        grid=(grid,),
        in_specs=[
            pl.BlockSpec((N, C, _BH, W), lambda i: (0, 0, i, 0)),
            pl.BlockSpec((N, _BH, W), lambda i: (0, i, 0)),
            pl.BlockSpec((N, _BH, W), lambda i: (0, i, 0)),
            pl.BlockSpec((N, _BH, W), lambda i: (0, i, 0)),
            pl.BlockSpec((N, _BH, W), lambda i: (0, i, 0)),
        ],
        out_specs=pl.BlockSpec((1, 128), lambda i: (0, 0)),
        out_shape=jax.ShapeDtypeStruct((1, 128), jnp.float32),
    )(pred, shrink_map, shrink_mask, threshold_map, threshold_mask)

    hist = _sc_hist()(pred, shrink_map, shrink_mask)

    out = pl.pallas_call(
        _finalize_body,
        out_shape=jax.ShapeDtypeStruct((1, 128), jnp.float32),
    )(acc, hist)

    return (out[0, 0], out[0, 1], out[0, 2], out[0, 3])
